# baseline (speedup 1.0000x reference)
import functools

import jax
import jax.numpy as jnp
from jax import lax
from jax.experimental import pallas as pl
from jax.experimental.pallas import tpu as pltpu
from jax.experimental.pallas import tpu_sc as plsc

CAT_DIMS = (1000, 100000, 10000, 48, 2)
EMB_DIMS = (4, 21, 1, 1, 2)
OFFSETS = (0, 4, 25, 26, 27)
OUT_DIM = 29
BATCH = 16384

NC = 2
NS = 16
L = 16
NW = NC * NS
B_PER_W = BATCH // NW
NBLK = 4
BLK = B_PER_W // NBLK
NCHUNK = B_PER_W // L

V1 = CAT_DIMS[1]
D1 = EMB_DIMS[1]
K_MAX = (24 * V1) // 21 - 2
LAST8 = V1 - 8


def _clip(v, hi):
    return jnp.minimum(jnp.maximum(v, jnp.int32(0)), jnp.int32(hi))


def _body(x_hbm, t0_hbm, t1_hbm, t2_hbm, t3_hbm, t4_hbm, out_hbm,
          xs, klist, k2list, sbuf, win, w2, last8,
          st0, st3, st4, out_v, sem, sem2):
    wid = lax.axis_index("s") * NC + lax.axis_index("c")
    base = wid * B_PER_W

    small_descs = [
        pltpu.async_copy(t0_hbm, st0, sem2),
        pltpu.async_copy(t3_hbm, st3, sem2),
        pltpu.async_copy(t4_hbm, st4, sem2),
        pltpu.async_copy(t1_hbm.at[pl.ds(LAST8, 8), :], last8, sem2),
    ]

    pltpu.sync_copy(x_hbm.at[pl.ds(base, B_PER_W), :], xs)

    iota = lax.iota(jnp.int32, L)
    iota2 = jnp.int32(2) * iota
    c1 = jnp.full((L,), 1, jnp.int32)
    c2 = jnp.full((L,), 2, jnp.int32)

    for k in range(NCHUNK):
        j, o = k // 8, (k % 8) * L
        rows16 = jnp.int32(k * L) + iota
        i = _clip(plsc.load_gather(xs, [rows16, c1]), V1 - 1)
        kk = jnp.minimum((i * jnp.int32(8)) // jnp.int32(7), jnp.int32(K_MAX))
        sbuf[j, pl.ds(o, L)] = i * jnp.int32(24) - kk * jnp.int32(21)
        m = k // 4
        row_m = jnp.full((L,), m, jnp.int32)
        c0 = jnp.int32(32 * (k % 4)) + iota2
        plsc.store_scatter(klist, [row_m, c0], kk)
        plsc.store_scatter(klist, [row_m, c0 + jnp.int32(1)],
                           kk + jnp.int32(1))
        i2 = _clip(plsc.load_gather(xs, [rows16, c2]), CAT_DIMS[2] - 1)
        k2list[j, pl.ds(o, L)] = i2 * jnp.int32(8)

    descs = []
    for m in range(2 * NBLK):
        descs.append(
            pltpu.async_copy(
                t1_hbm.at[klist.at[m]], win.at[pl.ds(m * BLK, BLK)], sem))
    for j in range(NBLK):
        descs.append(
            pltpu.async_copy(
                t2_hbm.at[k2list.at[j]], w2.at[pl.ds(j * BLK, BLK)], sem))
    for dsc in descs:
        dsc.wait()
    for dsc in small_descs:
        dsc.wait()

    iota18 = jnp.int32(18) * iota
    c7 = jnp.full((L,), 7, jnp.int32)
    for k in range(NCHUNK):
        j, o = k // 8, (k % 8) * L
        rows16 = jnp.int32(k * L) + iota
        m = k // 4

        s = sbuf[j, pl.ds(o, L)]
        i1 = plsc.load_gather(xs, [rows16, c1])
        psel = i1 >= jnp.int32(V1 - 1)
        rowsel = rows16 + jnp.int32(64 * m)
        colbase = jnp.int32(18 * ((k % 4) * L)) + iota18 + s
        for c in range(D1):
            val = plsc.load_gather(win, [rowsel, colbase + jnp.int32(c)])
            patch = plsc.load_gather(
                last8, [c7, jnp.full((L,), c, jnp.int32)], mask=psel)
            val = jnp.where(psel, patch, val)
            col = jnp.full((L,), OFFSETS[1] + c, jnp.int32)
            plsc.store_scatter(out_v, [rows16, col], val)

        rr = rows16 - jnp.int32(128 * j)
        r2row = jnp.int32(128 * j) + lax.shift_right_logical(rr, 3)
        r2col = jnp.bitwise_and(rr, jnp.int32(7))
        val = plsc.load_gather(w2, [r2row, r2col])
        col = jnp.full((L,), OFFSETS[2], jnp.int32)
        plsc.store_scatter(out_v, [rows16, col], val)

        for t, st in ((0, st0), (3, st3), (4, st4)):
            iv = _clip(
                plsc.load_gather(xs, [rows16, jnp.full((L,), t, jnp.int32)]),
                CAT_DIMS[t] - 1)
            for c in range(EMB_DIMS[t]):
                val = plsc.load_gather(st, [iv, jnp.full((L,), c, jnp.int32)])
                col = jnp.full((L,), OFFSETS[t] + c, jnp.int32)
                plsc.store_scatter(out_v, [rows16, col], val)

    pltpu.sync_copy(out_v, out_hbm.at[pl.ds(base, B_PER_W), :])


@jax.jit
def kernel(x, emb0, emb1, emb2, emb3, emb4):
    mesh = plsc.VectorSubcoreMesh(
        core_axis_name="c", subcore_axis_name="s", num_cores=NC, num_subcores=NS
    )
    scratch = [
        pltpu.VMEM((B_PER_W, 5), jnp.int32),
        pltpu.VMEM((2 * NBLK, BLK), jnp.int32),
        pltpu.VMEM((NBLK, BLK), jnp.int32),
        pltpu.VMEM((NBLK, BLK), jnp.int32),
        pltpu.VMEM((2 * B_PER_W, D1), jnp.float32),
        pltpu.VMEM((B_PER_W, 1), jnp.float32),
        pltpu.VMEM((8, D1), jnp.float32),
        pltpu.VMEM((CAT_DIMS[0], EMB_DIMS[0]), jnp.float32),
        pltpu.VMEM((CAT_DIMS[3], EMB_DIMS[3]), jnp.float32),
        pltpu.VMEM((CAT_DIMS[4], EMB_DIMS[4]), jnp.float32),
        pltpu.VMEM((B_PER_W, OUT_DIM), jnp.float32),
        pltpu.SemaphoreType.DMA,
        pltpu.SemaphoreType.DMA,
    ]
    fn = pl.kernel(
        _body,
        out_type=jax.ShapeDtypeStruct((BATCH, OUT_DIM), jnp.float32),
        mesh=mesh,
        scratch_types=scratch,
        compiler_params=pltpu.CompilerParams(
            use_tc_tiling_on_sc=False, needs_layout_passes=False
        ),
    )
    return fn(x, emb0, emb1, emb2, emb3, emb4)


# speedup vs baseline: 1.4937x; 1.4937x over previous
import functools

import jax
import jax.numpy as jnp
from jax import lax
from jax.experimental import pallas as pl
from jax.experimental.pallas import tpu as pltpu
from jax.experimental.pallas import tpu_sc as plsc

CAT_DIMS = (1000, 100000, 10000, 48, 2)
EMB_DIMS = (4, 21, 1, 1, 2)
OFFSETS = (0, 4, 25, 26, 27)
OUT_DIM = 29
BATCH = 16384

NC = 2
NS = 16
L = 16
NW = NC * NS
B_PER_W = BATCH // NW
NBLK = 4
BLK = B_PER_W // NBLK
NCHUNK = B_PER_W // L

V1 = CAT_DIMS[1]
D1 = EMB_DIMS[1]
K_MAX = (24 * V1) // 21 - 2
LAST8 = V1 - 8


def _clip(v, hi):
    return jnp.minimum(jnp.maximum(v, jnp.int32(0)), jnp.int32(hi))


def _body(x_hbm, t0_hbm, t1_hbm, t2_hbm, t3_hbm, t4_hbm, out_hbm,
          xs, klist, sbuf, win, last8,
          st0, st2, st3, st4, out_v, sem, sem2):
    wid = lax.axis_index("s") * NC + lax.axis_index("c")
    base = wid * B_PER_W

    small_descs = [
        pltpu.async_copy(t0_hbm, st0, sem2),
        pltpu.async_copy(t2_hbm, st2, sem2),
        pltpu.async_copy(t3_hbm, st3, sem2),
        pltpu.async_copy(t4_hbm, st4, sem2),
        pltpu.async_copy(t1_hbm.at[pl.ds(LAST8, 8), :], last8, sem2),
    ]

    pltpu.sync_copy(x_hbm.at[pl.ds(base, B_PER_W), :], xs)

    iota = lax.iota(jnp.int32, L)
    iota2 = jnp.int32(2) * iota
    c1 = jnp.full((L,), 1, jnp.int32)
    c2 = jnp.full((L,), 2, jnp.int32)

    for k in range(NCHUNK):
        j, o = k // 8, (k % 8) * L
        rows16 = jnp.int32(k * L) + iota
        i = _clip(plsc.load_gather(xs, [rows16, c1]), V1 - 1)
        kk = jnp.minimum((i * jnp.int32(8)) // jnp.int32(7), jnp.int32(K_MAX))
        sbuf[j, pl.ds(o, L)] = i * jnp.int32(24) - kk * jnp.int32(21)
        m = k // 4
        row_m = jnp.full((L,), m, jnp.int32)
        c0 = jnp.int32(32 * (k % 4)) + iota2
        plsc.store_scatter(klist, [row_m, c0], kk)
        plsc.store_scatter(klist, [row_m, c0 + jnp.int32(1)],
                           kk + jnp.int32(1))

    descs = []
    for m in range(2 * NBLK):
        descs.append(
            pltpu.async_copy(
                t1_hbm.at[klist.at[m]], win.at[pl.ds(m * BLK, BLK)], sem))
    for dsc in descs:
        dsc.wait()
    for dsc in small_descs:
        dsc.wait()

    iota18 = jnp.int32(18) * iota
    c7 = jnp.full((L,), 7, jnp.int32)
    for k in range(NCHUNK):
        j, o = k // 8, (k % 8) * L
        rows16 = jnp.int32(k * L) + iota
        m = k // 4

        s = sbuf[j, pl.ds(o, L)]
        i1 = plsc.load_gather(xs, [rows16, c1])
        psel = i1 >= jnp.int32(V1 - 1)
        rowsel = rows16 + jnp.int32(64 * m)
        colbase = jnp.int32(18 * ((k % 4) * L)) + iota18 + s
        for c in range(D1):
            val = plsc.load_gather(win, [rowsel, colbase + jnp.int32(c)])
            patch = plsc.load_gather(
                last8, [c7, jnp.full((L,), c, jnp.int32)], mask=psel)
            val = jnp.where(psel, patch, val)
            col = jnp.full((L,), OFFSETS[1] + c, jnp.int32)
            plsc.store_scatter(out_v, [rows16, col], val)

        i2 = _clip(plsc.load_gather(xs, [rows16, c2]), CAT_DIMS[2] - 1)
        val = plsc.load_gather(st2, [i2])
        col = jnp.full((L,), OFFSETS[2], jnp.int32)
        plsc.store_scatter(out_v, [rows16, col], val)

        for t, st in ((0, st0), (3, st3), (4, st4)):
            iv = _clip(
                plsc.load_gather(xs, [rows16, jnp.full((L,), t, jnp.int32)]),
                CAT_DIMS[t] - 1)
            for c in range(EMB_DIMS[t]):
                val = plsc.load_gather(st, [iv, jnp.full((L,), c, jnp.int32)])
                col = jnp.full((L,), OFFSETS[t] + c, jnp.int32)
                plsc.store_scatter(out_v, [rows16, col], val)

    pltpu.sync_copy(out_v, out_hbm.at[pl.ds(base, B_PER_W), :])


@jax.jit
def kernel(x, emb0, emb1, emb2, emb3, emb4):
    mesh = plsc.VectorSubcoreMesh(
        core_axis_name="c", subcore_axis_name="s", num_cores=NC, num_subcores=NS
    )
    scratch = [
        pltpu.VMEM((B_PER_W, 5), jnp.int32),
        pltpu.VMEM((2 * NBLK, BLK), jnp.int32),
        pltpu.VMEM((NBLK, BLK), jnp.int32),
        pltpu.VMEM((2 * B_PER_W, D1), jnp.float32),
        pltpu.VMEM((8, D1), jnp.float32),
        pltpu.VMEM((CAT_DIMS[0], EMB_DIMS[0]), jnp.float32),
        pltpu.VMEM((CAT_DIMS[2],), jnp.float32),
        pltpu.VMEM((CAT_DIMS[3], EMB_DIMS[3]), jnp.float32),
        pltpu.VMEM((CAT_DIMS[4], EMB_DIMS[4]), jnp.float32),
        pltpu.VMEM((B_PER_W, OUT_DIM), jnp.float32),
        pltpu.SemaphoreType.DMA,
        pltpu.SemaphoreType.DMA,
    ]
    fn = pl.kernel(
        _body,
        out_type=jax.ShapeDtypeStruct((BATCH, OUT_DIM), jnp.float32),
        mesh=mesh,
        scratch_types=scratch,
        compiler_params=pltpu.CompilerParams(
            use_tc_tiling_on_sc=False, needs_layout_passes=False
        ),
    )
    return fn(x, emb0, emb1, emb2.reshape(-1), emb3, emb4)


# speedup vs baseline: 1.5649x; 1.0477x over previous
import functools

import jax
import jax.numpy as jnp
from jax import lax
from jax.experimental import pallas as pl
from jax.experimental.pallas import tpu as pltpu
from jax.experimental.pallas import tpu_sc as plsc

CAT_DIMS = (1000, 100000, 10000, 48, 2)
EMB_DIMS = (4, 21, 1, 1, 2)
OFFSETS = (0, 4, 25, 26, 27)
OUT_DIM = 29
BATCH = 16384

NC = 2
NS = 16
L = 16
NW = NC * NS
B_PER_W = BATCH // NW
NBLK = 4
BLK = B_PER_W // NBLK
NCHUNK = B_PER_W // L

V1 = CAT_DIMS[1]
D1 = EMB_DIMS[1]
K_MAX = (24 * V1) // 21 - 2
LAST8 = V1 - 8


def _clip(v, hi):
    return jnp.minimum(jnp.maximum(v, jnp.int32(0)), jnp.int32(hi))


def _body(x_hbm, t0_hbm, t1_hbm, t2_hbm, t3_hbm, t4_hbm, out_hbm,
          i0, i1, i2, i3, i4, klist, sbuf, win, last8,
          st0, st2, st3, st4, out_v, sem, sem2):
    idxs = (i0, i1, i2, i3, i4)
    wid = lax.axis_index("s") * NC + lax.axis_index("c")
    base = wid * B_PER_W

    small_descs = [
        pltpu.async_copy(t0_hbm, st0, sem2),
        pltpu.async_copy(t2_hbm, st2, sem2),
        pltpu.async_copy(t3_hbm, st3, sem2),
        pltpu.async_copy(t4_hbm, st4, sem2),
        pltpu.async_copy(t1_hbm.at[pl.ds(LAST8, 8), :], last8, sem2),
    ]

    for t in range(5):
        pltpu.sync_copy(x_hbm.at[t, pl.ds(wid * NBLK, NBLK), :], idxs[t])

    iota = lax.iota(jnp.int32, L)
    iota2 = jnp.int32(2) * iota

    for k in range(NCHUNK):
        j, o = k // 8, (k % 8) * L
        rows16 = jnp.int32(k * L) + iota
        i = _clip(idxs[1][j, pl.ds(o, L)], V1 - 1)
        kk = jnp.minimum((i * jnp.int32(8)) // jnp.int32(7), jnp.int32(K_MAX))
        sbuf[j, pl.ds(o, L)] = i * jnp.int32(24) - kk * jnp.int32(21)
        m = k // 4
        row_m = jnp.full((L,), m, jnp.int32)
        c0 = jnp.int32(32 * (k % 4)) + iota2
        plsc.store_scatter(klist, [row_m, c0], kk)
        plsc.store_scatter(klist, [row_m, c0 + jnp.int32(1)],
                           kk + jnp.int32(1))

    descs = []
    for m in range(2 * NBLK):
        descs.append(
            pltpu.async_copy(
                t1_hbm.at[klist.at[m]], win.at[pl.ds(m * BLK, BLK)], sem))
    for dsc in descs:
        dsc.wait()
    for dsc in small_descs:
        dsc.wait()

    iota18 = jnp.int32(18) * iota
    c7 = jnp.full((L,), 7, jnp.int32)
    for k in range(NCHUNK):
        j, o = k // 8, (k % 8) * L
        rows16 = jnp.int32(k * L) + iota
        m = k // 4

        s = sbuf[j, pl.ds(o, L)]
        psel = idxs[1][j, pl.ds(o, L)] >= jnp.int32(V1 - 1)
        rowsel = rows16 + jnp.int32(64 * m)
        colbase = jnp.int32(18 * ((k % 4) * L)) + iota18 + s
        for c in range(D1):
            val = plsc.load_gather(win, [rowsel, colbase + jnp.int32(c)])
            patch = plsc.load_gather(
                last8, [c7, jnp.full((L,), c, jnp.int32)], mask=psel)
            val = jnp.where(psel, patch, val)
            col = jnp.full((L,), OFFSETS[1] + c, jnp.int32)
            plsc.store_scatter(out_v, [rows16, col], val)

        i2 = _clip(idxs[2][j, pl.ds(o, L)], CAT_DIMS[2] - 1)
        val = plsc.load_gather(st2, [i2])
        col = jnp.full((L,), OFFSETS[2], jnp.int32)
        plsc.store_scatter(out_v, [rows16, col], val)

        for t, st in ((0, st0), (3, st3), (4, st4)):
            iv = _clip(idxs[t][j, pl.ds(o, L)], CAT_DIMS[t] - 1)
            for c in range(EMB_DIMS[t]):
                val = plsc.load_gather(st, [iv, jnp.full((L,), c, jnp.int32)])
                col = jnp.full((L,), OFFSETS[t] + c, jnp.int32)
                plsc.store_scatter(out_v, [rows16, col], val)

    pltpu.sync_copy(out_v, out_hbm.at[pl.ds(base, B_PER_W), :])


@jax.jit
def kernel(x, emb0, emb1, emb2, emb3, emb4):
    mesh = plsc.VectorSubcoreMesh(
        core_axis_name="c", subcore_axis_name="s", num_cores=NC, num_subcores=NS
    )
    scratch = [
        pltpu.VMEM((NBLK, BLK), jnp.int32),
        pltpu.VMEM((NBLK, BLK), jnp.int32),
        pltpu.VMEM((NBLK, BLK), jnp.int32),
        pltpu.VMEM((NBLK, BLK), jnp.int32),
        pltpu.VMEM((NBLK, BLK), jnp.int32),
        pltpu.VMEM((2 * NBLK, BLK), jnp.int32),
        pltpu.VMEM((NBLK, BLK), jnp.int32),
        pltpu.VMEM((2 * B_PER_W, D1), jnp.float32),
        pltpu.VMEM((8, D1), jnp.float32),
        pltpu.VMEM((CAT_DIMS[0], EMB_DIMS[0]), jnp.float32),
        pltpu.VMEM((CAT_DIMS[2],), jnp.float32),
        pltpu.VMEM((CAT_DIMS[3], EMB_DIMS[3]), jnp.float32),
        pltpu.VMEM((CAT_DIMS[4], EMB_DIMS[4]), jnp.float32),
        pltpu.VMEM((B_PER_W, OUT_DIM), jnp.float32),
        pltpu.SemaphoreType.DMA,
        pltpu.SemaphoreType.DMA,
    ]
    fn = pl.kernel(
        _body,
        out_type=jax.ShapeDtypeStruct((BATCH, OUT_DIM), jnp.float32),
        mesh=mesh,
        scratch_types=scratch,
        compiler_params=pltpu.CompilerParams(
            use_tc_tiling_on_sc=False, needs_layout_passes=False
        ),
    )
    x_t = x.T.reshape(5, BATCH // BLK, BLK)
    return fn(x_t, emb0, emb1, emb2.reshape(-1), emb3, emb4)


# speedup vs baseline: 1.5832x; 1.0117x over previous
import functools

import jax
import jax.numpy as jnp
from jax import lax
from jax.experimental import pallas as pl
from jax.experimental.pallas import tpu as pltpu
from jax.experimental.pallas import tpu_sc as plsc

CAT_DIMS = (1000, 100000, 10000, 48, 2)
EMB_DIMS = (4, 21, 1, 1, 2)
OFFSETS = (0, 4, 25, 26, 27)
OUT_DIM = 29
BATCH = 16384

NC = 2
NS = 16
L = 16
NW = NC * NS
B_PER_W = BATCH // NW
NBLK = 4
BLK = B_PER_W // NBLK
NCHUNK = B_PER_W // L

V1 = CAT_DIMS[1]
D1 = EMB_DIMS[1]
K_MAX = (24 * V1) // 21 - 2
LAST8 = V1 - 8


def _clip(v, hi):
    return jnp.minimum(jnp.maximum(v, jnp.int32(0)), jnp.int32(hi))


def _body(x_hbm, t0_hbm, t1_hbm, t2_hbm, t3_hbm, t4_hbm, out_hbm,
          i0, i1, i2, i3, i4, klist, sbuf, win, last8,
          st0, st2, st3, st4, out_v, sem, sem2):
    idxs = (i0, i1, i2, i3, i4)
    wid = lax.axis_index("s") * NC + lax.axis_index("c")
    base = wid * B_PER_W

    small_descs = [
        pltpu.async_copy(t0_hbm, st0, sem2),
        pltpu.async_copy(t2_hbm, st2, sem2),
        pltpu.async_copy(t3_hbm, st3, sem2),
        pltpu.async_copy(t4_hbm, st4, sem2),
        pltpu.async_copy(t1_hbm.at[pl.ds(LAST8, 8), :], last8, sem2),
    ]

    for t in range(5):
        pltpu.sync_copy(x_hbm.at[t, pl.ds(wid * NBLK, NBLK), :], idxs[t])

    iota = lax.iota(jnp.int32, L)
    iota2 = jnp.int32(2) * iota

    descs = []
    for k in range(NCHUNK):
        j, o = k // 8, (k % 8) * L
        i = _clip(idxs[1][j, pl.ds(o, L)], V1 - 1)
        kk = jnp.minimum((i * jnp.int32(8)) // jnp.int32(7), jnp.int32(K_MAX))
        sbuf[j, pl.ds(o, L)] = i * jnp.int32(24) - kk * jnp.int32(21)
        m = k // 4
        row_m = jnp.full((L,), m, jnp.int32)
        c0 = jnp.int32(32 * (k % 4)) + iota2
        plsc.store_scatter(klist, [row_m, c0], kk)
        plsc.store_scatter(klist, [row_m, c0 + jnp.int32(1)],
                           kk + jnp.int32(1))
        if k % 4 == 3:
            descs.append(
                pltpu.async_copy(
                    t1_hbm.at[klist.at[m]],
                    win.at[pl.ds(m * BLK, BLK)], sem))

    for dsc in small_descs:
        dsc.wait()
    for k in range(NCHUNK):
        j, o = k // 8, (k % 8) * L
        rows16 = jnp.int32(k * L) + iota

        i2 = _clip(idxs[2][j, pl.ds(o, L)], CAT_DIMS[2] - 1)
        val = plsc.load_gather(st2, [i2])
        col = jnp.full((L,), OFFSETS[2], jnp.int32)
        plsc.store_scatter(out_v, [rows16, col], val)

        for t, st in ((0, st0), (3, st3), (4, st4)):
            iv = _clip(idxs[t][j, pl.ds(o, L)], CAT_DIMS[t] - 1)
            for c in range(EMB_DIMS[t]):
                val = plsc.load_gather(st, [iv, jnp.full((L,), c, jnp.int32)])
                col = jnp.full((L,), OFFSETS[t] + c, jnp.int32)
                plsc.store_scatter(out_v, [rows16, col], val)

    iota18 = jnp.int32(18) * iota
    c7 = jnp.full((L,), 7, jnp.int32)
    for m in range(2 * NBLK):
        descs[m].wait()
        for k in range(4 * m, 4 * m + 4):
            j, o = k // 8, (k % 8) * L
            rows16 = jnp.int32(k * L) + iota

            s = sbuf[j, pl.ds(o, L)]
            psel = idxs[1][j, pl.ds(o, L)] >= jnp.int32(V1 - 1)
            rowsel = rows16 + jnp.int32(64 * m)
            colbase = jnp.int32(18 * ((k % 4) * L)) + iota18 + s
            for c in range(D1):
                val = plsc.load_gather(win, [rowsel, colbase + jnp.int32(c)])
                patch = plsc.load_gather(
                    last8, [c7, jnp.full((L,), c, jnp.int32)], mask=psel)
                val = jnp.where(psel, patch, val)
                col = jnp.full((L,), OFFSETS[1] + c, jnp.int32)
                plsc.store_scatter(out_v, [rows16, col], val)

    pltpu.sync_copy(out_v, out_hbm.at[pl.ds(base, B_PER_W), :])


@jax.jit
def kernel(x, emb0, emb1, emb2, emb3, emb4):
    mesh = plsc.VectorSubcoreMesh(
        core_axis_name="c", subcore_axis_name="s", num_cores=NC, num_subcores=NS
    )
    scratch = [
        pltpu.VMEM((NBLK, BLK), jnp.int32),
        pltpu.VMEM((NBLK, BLK), jnp.int32),
        pltpu.VMEM((NBLK, BLK), jnp.int32),
        pltpu.VMEM((NBLK, BLK), jnp.int32),
        pltpu.VMEM((NBLK, BLK), jnp.int32),
        pltpu.VMEM((2 * NBLK, BLK), jnp.int32),
        pltpu.VMEM((NBLK, BLK), jnp.int32),
        pltpu.VMEM((2 * B_PER_W, D1), jnp.float32),
        pltpu.VMEM((8, D1), jnp.float32),
        pltpu.VMEM((CAT_DIMS[0], EMB_DIMS[0]), jnp.float32),
        pltpu.VMEM((CAT_DIMS[2],), jnp.float32),
        pltpu.VMEM((CAT_DIMS[3], EMB_DIMS[3]), jnp.float32),
        pltpu.VMEM((CAT_DIMS[4], EMB_DIMS[4]), jnp.float32),
        pltpu.VMEM((B_PER_W, OUT_DIM), jnp.float32),
        pltpu.SemaphoreType.DMA,
        pltpu.SemaphoreType.DMA,
    ]
    fn = pl.kernel(
        _body,
        out_type=jax.ShapeDtypeStruct((BATCH, OUT_DIM), jnp.float32),
        mesh=mesh,
        scratch_types=scratch,
        compiler_params=pltpu.CompilerParams(
            use_tc_tiling_on_sc=False, needs_layout_passes=False
        ),
    )
    x_t = x.T.reshape(5, BATCH // BLK, BLK)
    return fn(x_t, emb0, emb1, emb2.reshape(-1), emb3, emb4)
